# baseline (speedup 1.0000x reference)
import functools

import jax
import jax.numpy as jnp
from jax import lax
from jax.experimental import pallas as pl
from jax.experimental.pallas import tpu as pltpu
from jax.experimental.pallas import tpu_sc as plsc

N_NODES = 50000
N_EDGES = 800000
H = 64
HH = 32
L = 3

NTILES = 16
NP = 50176
RPT = NP // NTILES
ZCH = RPT // 8
EPT = 50176
EP = NTILES * EPT
EROWS = EP // 128
TROWS = EPT // 128
CROWS = EROWS // 2
CTROWS = CROWS // NTILES
RB = 1024
GRID = NP // RB

_SC_PARAMS = pltpu.CompilerParams(use_tc_tiling_on_sc=False)


def _edge_a_body(ei_hbm, g_hbm, za_hbm, zb_hbm,
                 idx_v, srcg_v, dstg_v, ra_v, rb_v, gsem, sasem, sbsem):
    c = lax.axis_index("c")
    s = lax.axis_index("s")

    offs = c * NP
    offd = 2 * NP + c * NP
    obase = c * EP + s * EPT

    def stage_fire(blk):
        slot = blk & 1
        pltpu.sync_copy(ei_hbm.at[s * TROWS + blk], idx_v.at[slot])
        for j in range(8):
            sj = pl.ds(j * 16, 16)
            srcg_v[slot, sj] = idx_v[slot, 0, sj] + offs
            dstg_v[slot, sj] = idx_v[slot, 1, sj] + offd
        sem = gsem.at[slot]
        pltpu.async_copy(g_hbm.at[srcg_v.at[slot]],
                         ra_v.at[pl.ds(slot * 128, 128)], sem)
        pltpu.async_copy(g_hbm.at[dstg_v.at[slot]],
                         rb_v.at[pl.ds(slot * 128, 128)], sem)

    @pl.loop(0, TROWS + 2)
    def _(blk):
        @pl.when(blk >= 2)
        def _():
            par = blk & 1
            pltpu.make_async_copy(za_hbm.at[pl.ds(0, 128)],
                                  ra_v.at[pl.ds(0, 128)],
                                  sasem.at[par]).wait()
            pltpu.make_async_copy(zb_hbm.at[pl.ds(0, 128)],
                                  rb_v.at[pl.ds(0, 128)],
                                  sbsem.at[par]).wait()

        @pl.when(blk < TROWS)
        def _():
            stage_fire(blk)

        @pl.when(jnp.logical_and(blk >= 1, blk <= TROWS))
        def _():
            k = blk - 1
            par = k & 1
            pltpu.make_async_copy(g_hbm.at[pl.ds(0, 256)],
                                  ra_v.at[pl.ds(0, 256)],
                                  gsem.at[par]).wait()
            orow = obase + k * 128
            pltpu.async_copy(ra_v.at[pl.ds(par * 128, 128)],
                             za_hbm.at[pl.ds(orow, 128)], sasem.at[par])
            pltpu.async_copy(rb_v.at[pl.ds(par * 128, 128)],
                             zb_hbm.at[pl.ds(orow, 128)], sbsem.at[par])


_edge_a = pl.kernel(
    _edge_a_body,
    out_type=[jax.ShapeDtypeStruct((2 * EP, HH), jnp.bfloat16),
              jax.ShapeDtypeStruct((2 * EP, HH), jnp.bfloat16)],
    mesh=plsc.VectorSubcoreMesh(core_axis_name="c", subcore_axis_name="s"),
    scratch_types=[
        pltpu.VMEM((2, 2, 128), jnp.int32),
        pltpu.VMEM((2, 128), jnp.int32),
        pltpu.VMEM((2, 128), jnp.int32),
        pltpu.VMEM((256, HH), jnp.bfloat16),
        pltpu.VMEM((256, HH), jnp.bfloat16),
        pltpu.SemaphoreType.DMA((2,)),
        pltpu.SemaphoreType.DMA((2,)),
        pltpu.SemaphoreType.DMA((2,)),
    ],
    compiler_params=_SC_PARAMS,
)


def _edge_b_body(ei_hbm, sin_hbm, out_hbm,
                 idx_v, sv_v, s_sh, lsem, ssem):
    c = lax.axis_index("c")
    s = lax.axis_index("s")

    zero = jnp.zeros((1, 16), jnp.float32)

    @pl.loop(0, 256)
    def _(i):
        for j in range(HH // 16):
            sv_v.at[pl.ds(i, 1), pl.ds(j * 16, 16)][...] = zero
    for k in range(12):
        pltpu.sync_copy(sv_v, s_sh.at[pl.ds(s * RPT + k * 256, 256)])
    pltpu.sync_copy(sv_v.at[pl.ds(0, RPT - 3072)],
                    s_sh.at[pl.ds(s * RPT + 3072, RPT - 3072)])
    plsc.subcore_barrier()

    ibase = c * EP + s * EPT

    @pl.loop(0, TROWS + 1)
    def _(blk):
        @pl.when(blk >= 2)
        def _():
            pltpu.make_async_copy(sin_hbm.at[pl.ds(0, 128)],
                                  sv_v.at[pl.ds(0, 128)],
                                  ssem.at[blk & 1]).wait()

        @pl.when(blk < TROWS)
        def _():
            slot = blk & 1
            pltpu.sync_copy(ei_hbm.at[s * TROWS + blk, 1], idx_v.at[slot])
            pltpu.async_copy(sin_hbm.at[pl.ds(ibase + blk * 128, 128)],
                             sv_v.at[pl.ds(slot * 128, 128)],
                             lsem.at[slot])

        @pl.when(jnp.logical_and(blk >= 1, blk <= TROWS))
        def _():
            k = blk - 1
            par = k & 1
            pltpu.make_async_copy(sin_hbm.at[pl.ds(0, 128)],
                                  sv_v.at[pl.ds(0, 128)],
                                  lsem.at[par]).wait()
            pltpu.async_copy(sv_v.at[pl.ds(par * 128, 128)],
                             s_sh.at[idx_v.at[par]],
                             ssem.at[par], add=True)

    pltpu.make_async_copy(sin_hbm.at[pl.ds(0, 128)],
                          sv_v.at[pl.ds(0, 128)],
                          ssem.at[(TROWS - 1) & 1]).wait()

    plsc.subcore_barrier()
    pltpu.sync_copy(s_sh.at[pl.ds(s * RPT, RPT)],
                    out_hbm.at[pl.ds(c * NP + s * RPT, RPT)])


_edge_b = pl.kernel(
    _edge_b_body,
    out_type=jax.ShapeDtypeStruct((2 * NP, HH), jnp.float32),
    mesh=plsc.VectorSubcoreMesh(core_axis_name="c", subcore_axis_name="s"),
    scratch_types=[
        pltpu.VMEM((2, 128), jnp.int32),
        pltpu.VMEM((256, HH), jnp.float32),
        pltpu.VMEM_SHARED((NP, HH), jnp.float32),
        pltpu.SemaphoreType.DMA((2,)),
        pltpu.SemaphoreType.DMA((2,)),
    ],
    compiler_params=_SC_PARAMS,
)


def _cnt_body(ei_hbm, out_hbm, idx_v, ones_v, z1_v, cnt_sh):
    c = lax.axis_index("c")
    s = lax.axis_index("s")
    one = jnp.ones((16,), jnp.float32)
    for j in range(8):
        ones_v[pl.ds(j * 16, 16)] = one
    zero = jnp.zeros((16,), jnp.float32)

    @pl.loop(0, RPT // 16)
    def _(i):
        z1_v[pl.ds(i * 16, 16)] = zero
    pltpu.sync_copy(z1_v, cnt_sh.at[pl.ds(s * RPT, RPT)])
    plsc.subcore_barrier()

    base_row = c * CROWS + s * CTROWS

    @pl.loop(0, CTROWS)
    def _(i):
        pltpu.sync_copy(ei_hbm.at[base_row + i, 1], idx_v.at[0])
        pltpu.sync_copy(ones_v, cnt_sh.at[idx_v.at[0]], add=True)

    plsc.subcore_barrier()
    pltpu.sync_copy(cnt_sh.at[pl.ds(s * RPT, RPT)],
                    out_hbm.at[pl.ds(c * NP + s * RPT, RPT)])


_cnt = pl.kernel(
    _cnt_body,
    out_type=jax.ShapeDtypeStruct((2 * NP,), jnp.float32),
    mesh=plsc.VectorSubcoreMesh(core_axis_name="c", subcore_axis_name="s"),
    scratch_types=[
        pltpu.VMEM((1, 128), jnp.int32),
        pltpu.VMEM((128,), jnp.float32),
        pltpu.VMEM((RPT,), jnp.float32),
        pltpu.VMEM_SHARED((NP,), jnp.float32),
    ],
    compiler_params=_SC_PARAMS,
)


def _prep_body(x_ref, wp_ref, bp_ref, wa_ref, wb_ref, h_ref, g_ref):
    x = x_ref[...]
    h = jnp.dot(x, wp_ref[...], preferred_element_type=jnp.float32) + bp_ref[...]
    h_ref[...] = h
    ga = jnp.dot(h, wa_ref[...], preferred_element_type=jnp.float32)
    gb = jnp.dot(h, wb_ref[...], preferred_element_type=jnp.float32)
    g_ref[0] = ga[:, :HH].astype(jnp.bfloat16)
    g_ref[1] = ga[:, HH:].astype(jnp.bfloat16)
    g_ref[2] = gb[:, :HH].astype(jnp.bfloat16)
    g_ref[3] = gb[:, HH:].astype(jnp.bfloat16)


def _update_body(last, *refs):
    if last:
        (h_ref, s_ref, cnt_ref, we2_ref, be2_ref, wn1a_ref, wn1b_ref,
         bn1_ref, wn2_ref, bn2_ref, h2_ref) = refs
    else:
        (h_ref, s_ref, cnt_ref, we2_ref, be2_ref, wn1a_ref, wn1b_ref,
         bn1_ref, wn2_ref, bn2_ref, wa_ref, wb_ref, h2_ref, g_ref) = refs
    h = h_ref[...]
    ssum = jnp.concatenate([s_ref[0], s_ref[1]], axis=1)
    cnt = cnt_ref[0] + cnt_ref[1]
    agg = (jnp.dot(ssum, we2_ref[...], preferred_element_type=jnp.float32)
           + cnt * be2_ref[...])
    pre = (jnp.dot(h, wn1a_ref[...], preferred_element_type=jnp.float32)
           + jnp.dot(agg, wn1b_ref[...], preferred_element_type=jnp.float32)
           + bn1_ref[...])
    t = pre * jax.nn.sigmoid(pre)
    h2 = h + jnp.dot(t, wn2_ref[...], preferred_element_type=jnp.float32) + bn2_ref[...]
    h2_ref[...] = h2
    if not last:
        ga = jnp.dot(h2, wa_ref[...], preferred_element_type=jnp.float32)
        gb = jnp.dot(h2, wb_ref[...], preferred_element_type=jnp.float32)
        g_ref[0] = ga[:, :HH].astype(jnp.bfloat16)
        g_ref[1] = ga[:, HH:].astype(jnp.bfloat16)
        g_ref[2] = gb[:, :HH].astype(jnp.bfloat16)
        g_ref[3] = gb[:, HH:].astype(jnp.bfloat16)


def _esilu_body(za_ref, zb_ref, bt_ref, s_ref):
    z = (za_ref[...].astype(jnp.float32) + zb_ref[...].astype(jnp.float32)
         + bt_ref[...][:, None, :])
    s_ref[...] = z * jax.nn.sigmoid(z)


def _logits_body(h_ref, wr1_ref, br1_ref, wr2_ref, br2_ref, o_ref):
    h = h_ref[...]
    pre = jnp.dot(h, wr1_ref[...], preferred_element_type=jnp.float32) + br1_ref[...]
    t = pre * jax.nn.sigmoid(pre)
    o_ref[...] = jnp.sum(t * wr2_ref[...], axis=1, keepdims=True) + br2_ref[...]


def _softmax_body(l_ref, o_ref):
    r = lax.broadcasted_iota(jnp.int32, (NP // 128, 128), 0)
    cidx = lax.broadcasted_iota(jnp.int32, (NP // 128, 128), 1)
    flat = r * 128 + cidx
    logits = jnp.where(flat < N_NODES, l_ref[...], -jnp.inf)
    m = jnp.max(jnp.max(logits, axis=1, keepdims=True), axis=0, keepdims=True)
    e = jnp.exp(logits - m)
    ssum = jnp.sum(jnp.sum(e, axis=1, keepdims=True), axis=0, keepdims=True)
    o_ref[...] = e / ssum


_WSPEC = pl.BlockSpec((H, H), lambda i: (0, 0))
_BSPEC = pl.BlockSpec((1, H), lambda i: (0, 0))
_HSPEC = pl.BlockSpec((RB, H), lambda i: (i, 0))
_GSPEC = pl.BlockSpec((4, RB, HH), lambda i: (0, i, 0))

_prep = pl.pallas_call(
    _prep_body,
    grid=(GRID,),
    in_specs=[pl.BlockSpec((RB, 8), lambda i: (i, 0)),
              pl.BlockSpec((8, H), lambda i: (0, 0)),
              _BSPEC, _WSPEC, _WSPEC],
    out_specs=[_HSPEC, _GSPEC],
    out_shape=[jax.ShapeDtypeStruct((NP, H), jnp.float32),
               jax.ShapeDtypeStruct((4, NP, HH), jnp.bfloat16)],
)

_update_specs = [
    _HSPEC,
    pl.BlockSpec((2, RB, HH), lambda i: (0, i, 0)),
    pl.BlockSpec((2, RB, 1), lambda i: (0, i, 0)),
    _WSPEC, _BSPEC, _WSPEC, _WSPEC, _BSPEC, _WSPEC, _BSPEC,
]

_update_mid = pl.pallas_call(
    functools.partial(_update_body, False),
    grid=(GRID,),
    in_specs=_update_specs + [_WSPEC, _WSPEC],
    out_specs=[_HSPEC, _GSPEC],
    out_shape=[jax.ShapeDtypeStruct((NP, H), jnp.float32),
               jax.ShapeDtypeStruct((4, NP, HH), jnp.bfloat16)],
)

_update_last = pl.pallas_call(
    functools.partial(_update_body, True),
    grid=(GRID,),
    in_specs=_update_specs,
    out_specs=[_HSPEC],
    out_shape=[jax.ShapeDtypeStruct((NP, H), jnp.float32)],
)

ERB = 2048
EGRID = EP // 4 // ERB

_ESPEC = pl.BlockSpec((2, ERB, 128), lambda i: (0, i, 0))

_esilu = pl.pallas_call(
    _esilu_body,
    grid=(EGRID,),
    in_specs=[_ESPEC, _ESPEC, pl.BlockSpec((2, 128), lambda i: (0, 0))],
    out_specs=[_ESPEC],
    out_shape=[jax.ShapeDtypeStruct((2, EP // 4, 128), jnp.float32)],
)

_logits = pl.pallas_call(
    _logits_body,
    grid=(GRID,),
    in_specs=[_HSPEC, _WSPEC, _BSPEC, _BSPEC,
              pl.BlockSpec((1, 1), lambda i: (0, 0))],
    out_specs=[pl.BlockSpec((RB, 1), lambda i: (i, 0))],
    out_shape=[jax.ShapeDtypeStruct((NP, 1), jnp.float32)],
)

_softmax = pl.pallas_call(
    _softmax_body,
    out_shape=jax.ShapeDtypeStruct((NP // 128, 128), jnp.float32),
)


def kernel(node_feats, edge_index, Wp, bp, We1, be1, We2, be2,
           Wn1, bn1, Wn2, bn2, Wr1, br1, Wr2, br2):
    f32 = jnp.float32
    xp = jnp.zeros((NP, 8), f32).at[:N_NODES, :5].set(node_feats)
    wp8 = jnp.zeros((8, H), f32).at[:5].set(Wp)
    src = edge_index[0]
    dst = edge_index[1]
    pad = (N_NODES
           + (jnp.arange(EP - N_EDGES, dtype=jnp.int32) % (NP - N_NODES)))
    srcp = jnp.concatenate([src, pad]).reshape(EROWS, 128)
    dstp = jnp.concatenate([dst, pad]).reshape(EROWS, 128)
    ei = jnp.stack([srcp, dstp], axis=1)

    cnt = _cnt(ei).reshape(2, NP, 1)

    h, g = _prep(xp, wp8, bp.reshape(1, H), We1[0][:H], We1[0][H:])

    for l in range(L):
        za, zb = _edge_a(ei, g.reshape(4 * NP, HH))
        (sstr,) = _esilu(za.reshape(2, EP // 4, 128),
                         zb.reshape(2, EP // 4, 128),
                         jnp.tile(be1[l].reshape(2, HH), (1, 4)))
        s_acc = _edge_b(ei, sstr.reshape(2 * EP, HH))
        s3 = s_acc.reshape(2, NP, HH)
        args = (h, s3, cnt, We2[l], be2[l].reshape(1, H),
                Wn1[l][:H], Wn1[l][H:], bn1[l].reshape(1, H),
                Wn2[l], bn2[l].reshape(1, H))
        if l < L - 1:
            h, g = _update_mid(*args, We1[l + 1][:H], We1[l + 1][H:])
        else:
            (h,) = _update_last(*args)

    (logits,) = _logits(h, Wr1, br1.reshape(1, H), Wr2.reshape(1, H),
                        br2.reshape(1, 1))
    probs = _softmax(logits.reshape(NP // 128, 128))
    return probs.reshape(NP)[:N_NODES]


# speedup vs baseline: 1.6291x; 1.6291x over previous
import functools

import jax
import jax.numpy as jnp
from jax import lax
from jax.experimental import pallas as pl
from jax.experimental.pallas import tpu as pltpu
from jax.experimental.pallas import tpu_sc as plsc

N_NODES = 50000
N_EDGES = 800000
H = 64
HH = 32
L = 3

NTILES = 16
NP = 50176
RPT = NP // NTILES
ZCH = RPT // 8
EPT = 50176
EP = NTILES * EPT
EROWS = EP // 128
TROWS = EPT // 128
TR2 = TROWS // 2
CROWS = EROWS // 2
CTROWS = CROWS // NTILES
RB = 1024
GRID = NP // RB

_SC_PARAMS = pltpu.CompilerParams(use_tc_tiling_on_sc=False)


def _edge_a_body(ei_hbm, g_hbm, za_hbm, zb_hbm,
                 idx_v, srcg_v, dstg_v, ra_v, rb_v, gsem, sasem, sbsem):
    c = lax.axis_index("c")
    s = lax.axis_index("s")

    offs = c * NP
    offd = 2 * NP + c * NP
    obase = c * EP + s * EPT

    def stage_fire(blk):
        slot = blk & 1
        pltpu.sync_copy(ei_hbm.at[s * TROWS + blk], idx_v.at[slot])
        for j in range(8):
            sj = pl.ds(j * 16, 16)
            srcg_v[slot, sj] = idx_v[slot, 0, sj] + offs
            dstg_v[slot, sj] = idx_v[slot, 1, sj] + offd
        sem = gsem.at[slot]
        pltpu.async_copy(g_hbm.at[srcg_v.at[slot]],
                         ra_v.at[pl.ds(slot * 128, 128)], sem)
        pltpu.async_copy(g_hbm.at[dstg_v.at[slot]],
                         rb_v.at[pl.ds(slot * 128, 128)], sem)

    @pl.loop(0, TROWS + 2)
    def _(blk):
        @pl.when(blk >= 2)
        def _():
            par = blk & 1
            pltpu.make_async_copy(za_hbm.at[pl.ds(0, 128)],
                                  ra_v.at[pl.ds(0, 128)],
                                  sasem.at[par]).wait()
            pltpu.make_async_copy(zb_hbm.at[pl.ds(0, 128)],
                                  rb_v.at[pl.ds(0, 128)],
                                  sbsem.at[par]).wait()

        @pl.when(blk < TROWS)
        def _():
            stage_fire(blk)

        @pl.when(jnp.logical_and(blk >= 1, blk <= TROWS))
        def _():
            k = blk - 1
            par = k & 1
            pltpu.make_async_copy(g_hbm.at[pl.ds(0, 256)],
                                  ra_v.at[pl.ds(0, 256)],
                                  gsem.at[par]).wait()
            orow = obase + k * 128
            pltpu.async_copy(ra_v.at[pl.ds(par * 128, 128)],
                             za_hbm.at[pl.ds(orow, 128)], sasem.at[par])
            pltpu.async_copy(rb_v.at[pl.ds(par * 128, 128)],
                             zb_hbm.at[pl.ds(orow, 128)], sbsem.at[par])


_edge_a = pl.kernel(
    _edge_a_body,
    out_type=[jax.ShapeDtypeStruct((2 * EP, HH), jnp.float32),
              jax.ShapeDtypeStruct((2 * EP, HH), jnp.float32)],
    mesh=plsc.VectorSubcoreMesh(core_axis_name="c", subcore_axis_name="s"),
    scratch_types=[
        pltpu.VMEM((2, 2, 128), jnp.int32),
        pltpu.VMEM((2, 128), jnp.int32),
        pltpu.VMEM((2, 128), jnp.int32),
        pltpu.VMEM((256, HH), jnp.float32),
        pltpu.VMEM((256, HH), jnp.float32),
        pltpu.SemaphoreType.DMA((2,)),
        pltpu.SemaphoreType.DMA((2,)),
        pltpu.SemaphoreType.DMA((2,)),
    ],
    compiler_params=_SC_PARAMS,
)


def _edge_b_body(ei_hbm, sin_hbm, out_hbm,
                 idx_v, sv_v, s_sh, lsem, ssem):
    c = lax.axis_index("c")
    s = lax.axis_index("s")

    zero = jnp.zeros((1, 16), jnp.float32)

    @pl.loop(0, 256)
    def _(i):
        for j in range(HH // 16):
            sv_v.at[pl.ds(i, 1), pl.ds(j * 16, 16)][...] = zero
    for k in range(12):
        pltpu.sync_copy(sv_v.at[pl.ds(0, 256)],
                        s_sh.at[pl.ds(s * RPT + k * 256, 256)])
    pltpu.sync_copy(sv_v.at[pl.ds(0, RPT - 3072)],
                    s_sh.at[pl.ds(s * RPT + 3072, RPT - 3072)])
    plsc.subcore_barrier()

    ibase = c * EP + s * EPT

    @pl.loop(0, TR2 + 1)
    def _(i):
        @pl.when(i >= 2)
        def _():
            pltpu.make_async_copy(sin_hbm.at[pl.ds(0, 256)],
                                  sv_v.at[pl.ds(0, 256)],
                                  ssem.at[i & 1]).wait()

        @pl.when(i < TR2)
        def _():
            slot = i & 1
            pltpu.sync_copy(ei_hbm.at[s * TROWS + 2 * i, 1],
                            idx_v.at[slot, 0])
            pltpu.sync_copy(ei_hbm.at[s * TROWS + 2 * i + 1, 1],
                            idx_v.at[slot, 1])
            pltpu.async_copy(sin_hbm.at[pl.ds(ibase + i * 256, 256)],
                             sv_v.at[pl.ds(slot * 256, 256)],
                             lsem.at[slot])

        @pl.when(jnp.logical_and(i >= 1, i <= TR2))
        def _():
            k = i - 1
            par = k & 1
            pltpu.make_async_copy(sin_hbm.at[pl.ds(0, 256)],
                                  sv_v.at[pl.ds(0, 256)],
                                  lsem.at[par]).wait()
            pltpu.async_copy(sv_v.at[pl.ds(par * 256, 128)],
                             s_sh.at[idx_v.at[par, 0]],
                             ssem.at[par], add=True)
            pltpu.async_copy(sv_v.at[pl.ds(par * 256 + 128, 128)],
                             s_sh.at[idx_v.at[par, 1]],
                             ssem.at[par], add=True)

    pltpu.make_async_copy(sin_hbm.at[pl.ds(0, 256)],
                          sv_v.at[pl.ds(0, 256)],
                          ssem.at[(TR2 - 1) & 1]).wait()

    plsc.subcore_barrier()
    pltpu.sync_copy(s_sh.at[pl.ds(s * RPT, RPT)],
                    out_hbm.at[pl.ds(c * NP + s * RPT, RPT)])


_edge_b = pl.kernel(
    _edge_b_body,
    out_type=jax.ShapeDtypeStruct((2 * NP, HH), jnp.float32),
    mesh=plsc.VectorSubcoreMesh(core_axis_name="c", subcore_axis_name="s"),
    scratch_types=[
        pltpu.VMEM((2, 2, 128), jnp.int32),
        pltpu.VMEM((512, HH), jnp.float32),
        pltpu.VMEM_SHARED((NP, HH), jnp.float32),
        pltpu.SemaphoreType.DMA((2,)),
        pltpu.SemaphoreType.DMA((2,)),
    ],
    compiler_params=_SC_PARAMS,
)


def _cnt_body(ei_hbm, out_hbm, idx_v, ones_v, z1_v, cnt_sh):
    c = lax.axis_index("c")
    s = lax.axis_index("s")
    one = jnp.ones((16,), jnp.float32)
    for j in range(8):
        ones_v[pl.ds(j * 16, 16)] = one
    zero = jnp.zeros((16,), jnp.float32)

    @pl.loop(0, RPT // 16)
    def _(i):
        z1_v[pl.ds(i * 16, 16)] = zero
    pltpu.sync_copy(z1_v, cnt_sh.at[pl.ds(s * RPT, RPT)])
    plsc.subcore_barrier()

    base_row = c * CROWS + s * CTROWS

    @pl.loop(0, CTROWS)
    def _(i):
        pltpu.sync_copy(ei_hbm.at[base_row + i, 1], idx_v.at[0])
        pltpu.sync_copy(ones_v, cnt_sh.at[idx_v.at[0]], add=True)

    plsc.subcore_barrier()
    pltpu.sync_copy(cnt_sh.at[pl.ds(s * RPT, RPT)],
                    out_hbm.at[pl.ds(c * NP + s * RPT, RPT)])


_cnt = pl.kernel(
    _cnt_body,
    out_type=jax.ShapeDtypeStruct((2 * NP,), jnp.float32),
    mesh=plsc.VectorSubcoreMesh(core_axis_name="c", subcore_axis_name="s"),
    scratch_types=[
        pltpu.VMEM((1, 128), jnp.int32),
        pltpu.VMEM((128,), jnp.float32),
        pltpu.VMEM((RPT,), jnp.float32),
        pltpu.VMEM_SHARED((NP,), jnp.float32),
    ],
    compiler_params=_SC_PARAMS,
)


def _prep_body(x_ref, wp_ref, bp_ref, wa_ref, wb_ref, h_ref, g_ref):
    x = x_ref[...]
    h = jnp.dot(x, wp_ref[...], preferred_element_type=jnp.float32) + bp_ref[...]
    h_ref[...] = h
    ga = jnp.dot(h, wa_ref[...], preferred_element_type=jnp.float32)
    gb = jnp.dot(h, wb_ref[...], preferred_element_type=jnp.float32)
    g_ref[0] = ga[:, :HH]
    g_ref[1] = ga[:, HH:]
    g_ref[2] = gb[:, :HH]
    g_ref[3] = gb[:, HH:]


def _update_body(last, *refs):
    if last:
        (h_ref, s_ref, cnt_ref, we2_ref, be2_ref, wn1a_ref, wn1b_ref,
         bn1_ref, wn2_ref, bn2_ref, h2_ref) = refs
    else:
        (h_ref, s_ref, cnt_ref, we2_ref, be2_ref, wn1a_ref, wn1b_ref,
         bn1_ref, wn2_ref, bn2_ref, wa_ref, wb_ref, h2_ref, g_ref) = refs
    h = h_ref[...]
    ssum = jnp.concatenate([s_ref[0], s_ref[1]], axis=1)
    cnt = cnt_ref[0] + cnt_ref[1]
    agg = (jnp.dot(ssum, we2_ref[...], preferred_element_type=jnp.float32)
           + cnt * be2_ref[...])
    pre = (jnp.dot(h, wn1a_ref[...], preferred_element_type=jnp.float32)
           + jnp.dot(agg, wn1b_ref[...], preferred_element_type=jnp.float32)
           + bn1_ref[...])
    t = pre * jax.nn.sigmoid(pre)
    h2 = h + jnp.dot(t, wn2_ref[...], preferred_element_type=jnp.float32) + bn2_ref[...]
    h2_ref[...] = h2
    if not last:
        ga = jnp.dot(h2, wa_ref[...], preferred_element_type=jnp.float32)
        gb = jnp.dot(h2, wb_ref[...], preferred_element_type=jnp.float32)
        g_ref[0] = ga[:, :HH]
        g_ref[1] = ga[:, HH:]
        g_ref[2] = gb[:, :HH]
        g_ref[3] = gb[:, HH:]


def _esilu_body(za_ref, zb_ref, bt_ref, s_ref):
    z = za_ref[...] + zb_ref[...] + bt_ref[...][:, None, :]
    s_ref[...] = z * jax.nn.sigmoid(z)


def _logits_body(h_ref, wr1_ref, br1_ref, wr2_ref, br2_ref, o_ref):
    h = h_ref[...]
    pre = jnp.dot(h, wr1_ref[...], preferred_element_type=jnp.float32) + br1_ref[...]
    t = pre * jax.nn.sigmoid(pre)
    o_ref[...] = jnp.sum(t * wr2_ref[...], axis=1, keepdims=True) + br2_ref[...]


def _softmax_body(l_ref, o_ref):
    r = lax.broadcasted_iota(jnp.int32, (NP // 128, 128), 0)
    cidx = lax.broadcasted_iota(jnp.int32, (NP // 128, 128), 1)
    flat = r * 128 + cidx
    logits = jnp.where(flat < N_NODES, l_ref[...], -jnp.inf)
    m = jnp.max(jnp.max(logits, axis=1, keepdims=True), axis=0, keepdims=True)
    e = jnp.exp(logits - m)
    ssum = jnp.sum(jnp.sum(e, axis=1, keepdims=True), axis=0, keepdims=True)
    o_ref[...] = e / ssum


_WSPEC = pl.BlockSpec((H, H), lambda i: (0, 0))
_BSPEC = pl.BlockSpec((1, H), lambda i: (0, 0))
_HSPEC = pl.BlockSpec((RB, H), lambda i: (i, 0))
_GSPEC = pl.BlockSpec((4, RB, HH), lambda i: (0, i, 0))

_prep = pl.pallas_call(
    _prep_body,
    grid=(GRID,),
    in_specs=[pl.BlockSpec((RB, 8), lambda i: (i, 0)),
              pl.BlockSpec((8, H), lambda i: (0, 0)),
              _BSPEC, _WSPEC, _WSPEC],
    out_specs=[_HSPEC, _GSPEC],
    out_shape=[jax.ShapeDtypeStruct((NP, H), jnp.float32),
               jax.ShapeDtypeStruct((4, NP, HH), jnp.float32)],
)

_update_specs = [
    _HSPEC,
    pl.BlockSpec((2, RB, HH), lambda i: (0, i, 0)),
    pl.BlockSpec((2, RB, 1), lambda i: (0, i, 0)),
    _WSPEC, _BSPEC, _WSPEC, _WSPEC, _BSPEC, _WSPEC, _BSPEC,
]

_update_mid = pl.pallas_call(
    functools.partial(_update_body, False),
    grid=(GRID,),
    in_specs=_update_specs + [_WSPEC, _WSPEC],
    out_specs=[_HSPEC, _GSPEC],
    out_shape=[jax.ShapeDtypeStruct((NP, H), jnp.float32),
               jax.ShapeDtypeStruct((4, NP, HH), jnp.float32)],
)

_update_last = pl.pallas_call(
    functools.partial(_update_body, True),
    grid=(GRID,),
    in_specs=_update_specs,
    out_specs=[_HSPEC],
    out_shape=[jax.ShapeDtypeStruct((NP, H), jnp.float32)],
)

ERB = 2048
EGRID = EP // 4 // ERB

_ESPEC = pl.BlockSpec((2, ERB, 128), lambda i: (0, i, 0))

_esilu = pl.pallas_call(
    _esilu_body,
    grid=(EGRID,),
    in_specs=[_ESPEC, _ESPEC, pl.BlockSpec((2, 128), lambda i: (0, 0))],
    out_specs=[_ESPEC],
    out_shape=[jax.ShapeDtypeStruct((2, EP // 4, 128), jnp.float32)],
)

_logits = pl.pallas_call(
    _logits_body,
    grid=(GRID,),
    in_specs=[_HSPEC, _WSPEC, _BSPEC, _BSPEC,
              pl.BlockSpec((1, 1), lambda i: (0, 0))],
    out_specs=[pl.BlockSpec((RB, 1), lambda i: (i, 0))],
    out_shape=[jax.ShapeDtypeStruct((NP, 1), jnp.float32)],
)

_softmax = pl.pallas_call(
    _softmax_body,
    out_shape=jax.ShapeDtypeStruct((NP // 128, 128), jnp.float32),
)


def kernel(node_feats, edge_index, Wp, bp, We1, be1, We2, be2,
           Wn1, bn1, Wn2, bn2, Wr1, br1, Wr2, br2):
    f32 = jnp.float32
    xp = jnp.zeros((NP, 8), f32).at[:N_NODES, :5].set(node_feats)
    wp8 = jnp.zeros((8, H), f32).at[:5].set(Wp)
    src = edge_index[0]
    dst = edge_index[1]
    pad = (N_NODES
           + (jnp.arange(EP - N_EDGES, dtype=jnp.int32) % (NP - N_NODES)))
    srcp = jnp.concatenate([src, pad]).reshape(EROWS, 128)
    dstp = jnp.concatenate([dst, pad]).reshape(EROWS, 128)
    ei = jnp.stack([srcp, dstp], axis=1)

    cnt = _cnt(ei).reshape(2, NP, 1)

    h, g = _prep(xp, wp8, bp.reshape(1, H), We1[0][:H], We1[0][H:])

    for l in range(L):
        za, zb = _edge_a(ei, g.reshape(4 * NP, HH))
        (sstr,) = _esilu(za.reshape(2, EP // 4, 128),
                         zb.reshape(2, EP // 4, 128),
                         jnp.tile(be1[l].reshape(2, HH), (1, 4)))
        s_acc = _edge_b(ei, sstr.reshape(2 * EP, HH))
        s3 = s_acc.reshape(2, NP, HH)
        args = (h, s3, cnt, We2[l], be2[l].reshape(1, H),
                Wn1[l][:H], Wn1[l][H:], bn1[l].reshape(1, H),
                Wn2[l], bn2[l].reshape(1, H))
        if l < L - 1:
            h, g = _update_mid(*args, We1[l + 1][:H], We1[l + 1][H:])
        else:
            (h,) = _update_last(*args)

    (logits,) = _logits(h, Wr1, br1.reshape(1, H), Wr2.reshape(1, H),
                        br2.reshape(1, 1))
    probs = _softmax(logits.reshape(NP // 128, 128))
    return probs.reshape(NP)[:N_NODES]


# speedup vs baseline: 1.6437x; 1.0090x over previous
import functools

import jax
import jax.numpy as jnp
from jax import lax
from jax.experimental import pallas as pl
from jax.experimental.pallas import tpu as pltpu
from jax.experimental.pallas import tpu_sc as plsc

N_NODES = 50000
N_EDGES = 800000
H = 64
HH = 32
L = 3

NTILES = 16
NP = 50176
RPT = NP // NTILES
ZCH = RPT // 8
EPT = 50176
EP = NTILES * EPT
EROWS = EP // 128
TROWS = EPT // 128
TR2 = TROWS // 2
NCH = 2
TRC = TROWS // NCH
EPC = EP // NCH
EPTC = EPT // NCH
TR2C = TR2 // NCH
CROWS = EROWS // 2
CTROWS = CROWS // NTILES
RB = 1024
GRID = NP // RB

_SC_PARAMS = pltpu.CompilerParams(use_tc_tiling_on_sc=False)


def _edge_a_body(chunk, ei_hbm, g_hbm, za_hbm, zb_hbm,
                 idx_v, srcg_v, dstg_v, ra_v, rb_v, gsem, sasem, sbsem):
    c = lax.axis_index("c")
    s = lax.axis_index("s")

    offs = c * NP
    offd = 2 * NP + c * NP
    obase = c * EPC + s * EPTC

    def stage_fire(blk):
        slot = blk & 1
        pltpu.sync_copy(ei_hbm.at[s * TROWS + chunk * TRC + blk],
                        idx_v.at[slot])
        for j in range(8):
            sj = pl.ds(j * 16, 16)
            srcg_v[slot, sj] = idx_v[slot, 0, sj] + offs
            dstg_v[slot, sj] = idx_v[slot, 1, sj] + offd
        sem = gsem.at[slot]
        pltpu.async_copy(g_hbm.at[srcg_v.at[slot]],
                         ra_v.at[pl.ds(slot * 128, 128)], sem)
        pltpu.async_copy(g_hbm.at[dstg_v.at[slot]],
                         rb_v.at[pl.ds(slot * 128, 128)], sem)

    @pl.loop(0, TRC + 2)
    def _(blk):
        @pl.when(blk >= 2)
        def _():
            par = blk & 1
            pltpu.make_async_copy(za_hbm.at[pl.ds(0, 128)],
                                  ra_v.at[pl.ds(0, 128)],
                                  sasem.at[par]).wait()
            pltpu.make_async_copy(zb_hbm.at[pl.ds(0, 128)],
                                  rb_v.at[pl.ds(0, 128)],
                                  sbsem.at[par]).wait()

        @pl.when(blk < TRC)
        def _():
            stage_fire(blk)

        @pl.when(jnp.logical_and(blk >= 1, blk <= TRC))
        def _():
            k = blk - 1
            par = k & 1
            pltpu.make_async_copy(g_hbm.at[pl.ds(0, 256)],
                                  ra_v.at[pl.ds(0, 256)],
                                  gsem.at[par]).wait()
            orow = obase + k * 128
            pltpu.async_copy(ra_v.at[pl.ds(par * 128, 128)],
                             za_hbm.at[pl.ds(orow, 128)], sasem.at[par])
            pltpu.async_copy(rb_v.at[pl.ds(par * 128, 128)],
                             zb_hbm.at[pl.ds(orow, 128)], sbsem.at[par])


def _make_edge_a(chunk):
  return pl.kernel(
    functools.partial(_edge_a_body, chunk),
    out_type=[jax.ShapeDtypeStruct((2 * EPC, HH), jnp.float32),
              jax.ShapeDtypeStruct((2 * EPC, HH), jnp.float32)],
    mesh=plsc.VectorSubcoreMesh(core_axis_name="c", subcore_axis_name="s"),
    scratch_types=[
        pltpu.VMEM((2, 2, 128), jnp.int32),
        pltpu.VMEM((2, 128), jnp.int32),
        pltpu.VMEM((2, 128), jnp.int32),
        pltpu.VMEM((256, HH), jnp.float32),
        pltpu.VMEM((256, HH), jnp.float32),
        pltpu.SemaphoreType.DMA((2,)),
        pltpu.SemaphoreType.DMA((2,)),
        pltpu.SemaphoreType.DMA((2,)),
    ],
    compiler_params=_SC_PARAMS,
  )


_edge_a1 = _make_edge_a(0)
_edge_a2 = _make_edge_a(1)


def _edge_b_body(ei_hbm, s1_hbm, s2_hbm, out_hbm,
                 idx_v, sv_v, s_sh, lsem, ssem):
    c = lax.axis_index("c")
    s = lax.axis_index("s")

    zero = jnp.zeros((1, 16), jnp.float32)

    @pl.loop(0, 256)
    def _(i):
        for j in range(HH // 16):
            sv_v.at[pl.ds(i, 1), pl.ds(j * 16, 16)][...] = zero
    for k in range(12):
        pltpu.sync_copy(sv_v.at[pl.ds(0, 256)],
                        s_sh.at[pl.ds(s * RPT + k * 256, 256)])
    pltpu.sync_copy(sv_v.at[pl.ds(0, RPT - 3072)],
                    s_sh.at[pl.ds(s * RPT + 3072, RPT - 3072)])
    plsc.subcore_barrier()

    ibase = c * EPC + s * EPTC

    @pl.loop(0, TR2 + 1)
    def _(i):
        @pl.when(i >= 2)
        def _():
            pltpu.make_async_copy(s1_hbm.at[pl.ds(0, 256)],
                                  sv_v.at[pl.ds(0, 256)],
                                  ssem.at[i & 1]).wait()

        @pl.when(i < TR2)
        def _():
            slot = i & 1
            pltpu.sync_copy(ei_hbm.at[s * TROWS + 2 * i, 1],
                            idx_v.at[slot, 0])
            pltpu.sync_copy(ei_hbm.at[s * TROWS + 2 * i + 1, 1],
                            idx_v.at[slot, 1])

            @pl.when(i < TR2C)
            def _():
                pltpu.async_copy(s1_hbm.at[pl.ds(ibase + i * 256, 256)],
                                 sv_v.at[pl.ds(slot * 256, 256)],
                                 lsem.at[slot])

            @pl.when(i >= TR2C)
            def _():
                pltpu.async_copy(
                    s2_hbm.at[pl.ds(ibase + (i - TR2C) * 256, 256)],
                    sv_v.at[pl.ds(slot * 256, 256)],
                    lsem.at[slot])

        @pl.when(jnp.logical_and(i >= 1, i <= TR2))
        def _():
            k = i - 1
            par = k & 1
            pltpu.make_async_copy(s1_hbm.at[pl.ds(0, 256)],
                                  sv_v.at[pl.ds(0, 256)],
                                  lsem.at[par]).wait()
            pltpu.async_copy(sv_v.at[pl.ds(par * 256, 128)],
                             s_sh.at[idx_v.at[par, 0]],
                             ssem.at[par], add=True)
            pltpu.async_copy(sv_v.at[pl.ds(par * 256 + 128, 128)],
                             s_sh.at[idx_v.at[par, 1]],
                             ssem.at[par], add=True)

    pltpu.make_async_copy(s1_hbm.at[pl.ds(0, 256)],
                          sv_v.at[pl.ds(0, 256)],
                          ssem.at[(TR2 - 1) & 1]).wait()

    plsc.subcore_barrier()
    pltpu.sync_copy(s_sh.at[pl.ds(s * RPT, RPT)],
                    out_hbm.at[pl.ds(c * NP + s * RPT, RPT)])


_edge_b = pl.kernel(
    _edge_b_body,
    out_type=jax.ShapeDtypeStruct((2 * NP, HH), jnp.float32),
    mesh=plsc.VectorSubcoreMesh(core_axis_name="c", subcore_axis_name="s"),
    scratch_types=[
        pltpu.VMEM((2, 2, 128), jnp.int32),
        pltpu.VMEM((512, HH), jnp.float32),
        pltpu.VMEM_SHARED((NP, HH), jnp.float32),
        pltpu.SemaphoreType.DMA((2,)),
        pltpu.SemaphoreType.DMA((2,)),
    ],
    compiler_params=_SC_PARAMS,
)


def _cnt_body(ei_hbm, out_hbm, idx_v, ones_v, z1_v, cnt_sh):
    c = lax.axis_index("c")
    s = lax.axis_index("s")
    one = jnp.ones((16,), jnp.float32)
    for j in range(8):
        ones_v[pl.ds(j * 16, 16)] = one
    zero = jnp.zeros((16,), jnp.float32)

    @pl.loop(0, RPT // 16)
    def _(i):
        z1_v[pl.ds(i * 16, 16)] = zero
    pltpu.sync_copy(z1_v, cnt_sh.at[pl.ds(s * RPT, RPT)])
    plsc.subcore_barrier()

    base_row = c * CROWS + s * CTROWS

    @pl.loop(0, CTROWS)
    def _(i):
        pltpu.sync_copy(ei_hbm.at[base_row + i, 1], idx_v.at[0])
        pltpu.sync_copy(ones_v, cnt_sh.at[idx_v.at[0]], add=True)

    plsc.subcore_barrier()
    pltpu.sync_copy(cnt_sh.at[pl.ds(s * RPT, RPT)],
                    out_hbm.at[pl.ds(c * NP + s * RPT, RPT)])


_cnt = pl.kernel(
    _cnt_body,
    out_type=jax.ShapeDtypeStruct((2 * NP,), jnp.float32),
    mesh=plsc.VectorSubcoreMesh(core_axis_name="c", subcore_axis_name="s"),
    scratch_types=[
        pltpu.VMEM((1, 128), jnp.int32),
        pltpu.VMEM((128,), jnp.float32),
        pltpu.VMEM((RPT,), jnp.float32),
        pltpu.VMEM_SHARED((NP,), jnp.float32),
    ],
    compiler_params=_SC_PARAMS,
)


def _prep_body(x_ref, wp_ref, bp_ref, wa_ref, wb_ref, h_ref, g_ref):
    x = x_ref[...]
    h = jnp.dot(x, wp_ref[...], preferred_element_type=jnp.float32) + bp_ref[...]
    h_ref[...] = h
    ga = jnp.dot(h, wa_ref[...], preferred_element_type=jnp.float32)
    gb = jnp.dot(h, wb_ref[...], preferred_element_type=jnp.float32)
    g_ref[0] = ga[:, :HH]
    g_ref[1] = ga[:, HH:]
    g_ref[2] = gb[:, :HH]
    g_ref[3] = gb[:, HH:]


def _update_body(last, *refs):
    if last:
        (h_ref, s_ref, cnt_ref, we2_ref, be2_ref, wn1a_ref, wn1b_ref,
         bn1_ref, wn2_ref, bn2_ref, h2_ref) = refs
    else:
        (h_ref, s_ref, cnt_ref, we2_ref, be2_ref, wn1a_ref, wn1b_ref,
         bn1_ref, wn2_ref, bn2_ref, wa_ref, wb_ref, h2_ref, g_ref) = refs
    h = h_ref[...]
    ssum = jnp.concatenate([s_ref[0], s_ref[1]], axis=1)
    cnt = cnt_ref[0] + cnt_ref[1]
    agg = (jnp.dot(ssum, we2_ref[...], preferred_element_type=jnp.float32)
           + cnt * be2_ref[...])
    pre = (jnp.dot(h, wn1a_ref[...], preferred_element_type=jnp.float32)
           + jnp.dot(agg, wn1b_ref[...], preferred_element_type=jnp.float32)
           + bn1_ref[...])
    t = pre * jax.nn.sigmoid(pre)
    h2 = h + jnp.dot(t, wn2_ref[...], preferred_element_type=jnp.float32) + bn2_ref[...]
    h2_ref[...] = h2
    if not last:
        ga = jnp.dot(h2, wa_ref[...], preferred_element_type=jnp.float32)
        gb = jnp.dot(h2, wb_ref[...], preferred_element_type=jnp.float32)
        g_ref[0] = ga[:, :HH]
        g_ref[1] = ga[:, HH:]
        g_ref[2] = gb[:, :HH]
        g_ref[3] = gb[:, HH:]


def _esilu_body(za_ref, zb_ref, bt_ref, s_ref):
    z = za_ref[...] + zb_ref[...] + bt_ref[...][:, None, :]
    s_ref[...] = z * jax.nn.sigmoid(z)


def _logits_body(h_ref, wr1_ref, br1_ref, wr2_ref, br2_ref, o_ref):
    h = h_ref[...]
    pre = jnp.dot(h, wr1_ref[...], preferred_element_type=jnp.float32) + br1_ref[...]
    t = pre * jax.nn.sigmoid(pre)
    o_ref[...] = jnp.sum(t * wr2_ref[...], axis=1, keepdims=True) + br2_ref[...]


def _softmax_body(l_ref, o_ref):
    r = lax.broadcasted_iota(jnp.int32, (NP // 128, 128), 0)
    cidx = lax.broadcasted_iota(jnp.int32, (NP // 128, 128), 1)
    flat = r * 128 + cidx
    logits = jnp.where(flat < N_NODES, l_ref[...], -jnp.inf)
    m = jnp.max(jnp.max(logits, axis=1, keepdims=True), axis=0, keepdims=True)
    e = jnp.exp(logits - m)
    ssum = jnp.sum(jnp.sum(e, axis=1, keepdims=True), axis=0, keepdims=True)
    o_ref[...] = e / ssum


_WSPEC = pl.BlockSpec((H, H), lambda i: (0, 0))
_BSPEC = pl.BlockSpec((1, H), lambda i: (0, 0))
_HSPEC = pl.BlockSpec((RB, H), lambda i: (i, 0))
_GSPEC = pl.BlockSpec((4, RB, HH), lambda i: (0, i, 0))

_prep = pl.pallas_call(
    _prep_body,
    grid=(GRID,),
    in_specs=[pl.BlockSpec((RB, 8), lambda i: (i, 0)),
              pl.BlockSpec((8, H), lambda i: (0, 0)),
              _BSPEC, _WSPEC, _WSPEC],
    out_specs=[_HSPEC, _GSPEC],
    out_shape=[jax.ShapeDtypeStruct((NP, H), jnp.float32),
               jax.ShapeDtypeStruct((4, NP, HH), jnp.float32)],
)

_update_specs = [
    _HSPEC,
    pl.BlockSpec((2, RB, HH), lambda i: (0, i, 0)),
    pl.BlockSpec((2, RB, 1), lambda i: (0, i, 0)),
    _WSPEC, _BSPEC, _WSPEC, _WSPEC, _BSPEC, _WSPEC, _BSPEC,
]

_update_mid = pl.pallas_call(
    functools.partial(_update_body, False),
    grid=(GRID,),
    in_specs=_update_specs + [_WSPEC, _WSPEC],
    out_specs=[_HSPEC, _GSPEC],
    out_shape=[jax.ShapeDtypeStruct((NP, H), jnp.float32),
               jax.ShapeDtypeStruct((4, NP, HH), jnp.float32)],
)

_update_last = pl.pallas_call(
    functools.partial(_update_body, True),
    grid=(GRID,),
    in_specs=_update_specs,
    out_specs=[_HSPEC],
    out_shape=[jax.ShapeDtypeStruct((NP, H), jnp.float32)],
)

ERB = 2048
EGRID = EPC // 4 // ERB

_ESPEC = pl.BlockSpec((2, ERB, 128), lambda i: (0, i, 0))

_esilu = pl.pallas_call(
    _esilu_body,
    grid=(EGRID,),
    in_specs=[_ESPEC, _ESPEC, pl.BlockSpec((2, 128), lambda i: (0, 0))],
    out_specs=[_ESPEC],
    out_shape=[jax.ShapeDtypeStruct((2, EPC // 4, 128), jnp.float32)],
)

_logits = pl.pallas_call(
    _logits_body,
    grid=(GRID,),
    in_specs=[_HSPEC, _WSPEC, _BSPEC, _BSPEC,
              pl.BlockSpec((1, 1), lambda i: (0, 0))],
    out_specs=[pl.BlockSpec((RB, 1), lambda i: (i, 0))],
    out_shape=[jax.ShapeDtypeStruct((NP, 1), jnp.float32)],
)

_softmax = pl.pallas_call(
    _softmax_body,
    out_shape=jax.ShapeDtypeStruct((NP // 128, 128), jnp.float32),
)


def kernel(node_feats, edge_index, Wp, bp, We1, be1, We2, be2,
           Wn1, bn1, Wn2, bn2, Wr1, br1, Wr2, br2):
    f32 = jnp.float32
    xp = jnp.zeros((NP, 8), f32).at[:N_NODES, :5].set(node_feats)
    wp8 = jnp.zeros((8, H), f32).at[:5].set(Wp)
    src = edge_index[0]
    dst = edge_index[1]
    pad = (N_NODES
           + (jnp.arange(EP - N_EDGES, dtype=jnp.int32) % (NP - N_NODES)))
    srcp = jnp.concatenate([src, pad]).reshape(EROWS, 128)
    dstp = jnp.concatenate([dst, pad]).reshape(EROWS, 128)
    ei = jnp.stack([srcp, dstp], axis=1)

    cnt = _cnt(ei).reshape(2, NP, 1)

    h, g = _prep(xp, wp8, bp.reshape(1, H), We1[0][:H], We1[0][H:])

    for l in range(L):
        gflat = g.reshape(4 * NP, HH)
        bt = jnp.tile(be1[l].reshape(2, HH), (1, 4))
        za1, zb1 = _edge_a1(ei, gflat)
        za2, zb2 = _edge_a2(ei, gflat)
        (s1,) = _esilu(za1.reshape(2, EPC // 4, 128),
                       zb1.reshape(2, EPC // 4, 128), bt)
        (s2,) = _esilu(za2.reshape(2, EPC // 4, 128),
                       zb2.reshape(2, EPC // 4, 128), bt)
        s_acc = _edge_b(ei, s1.reshape(2 * EPC, HH), s2.reshape(2 * EPC, HH))
        s3 = s_acc.reshape(2, NP, HH)
        args = (h, s3, cnt, We2[l], be2[l].reshape(1, H),
                Wn1[l][:H], Wn1[l][H:], bn1[l].reshape(1, H),
                Wn2[l], bn2[l].reshape(1, H))
        if l < L - 1:
            h, g = _update_mid(*args, We1[l + 1][:H], We1[l + 1][H:])
        else:
            (h,) = _update_last(*args)

    (logits,) = _logits(h, Wr1, br1.reshape(1, H), Wr2.reshape(1, H),
                        br2.reshape(1, 1))
    probs = _softmax(logits.reshape(NP // 128, 128))
    return probs.reshape(NP)[:N_NODES]


# speedup vs baseline: 1.7697x; 1.0766x over previous
import functools

import jax
import jax.numpy as jnp
from jax import lax
from jax.experimental import pallas as pl
from jax.experimental.pallas import tpu as pltpu
from jax.experimental.pallas import tpu_sc as plsc

N_NODES = 50000
N_EDGES = 800000
H = 64
HH = 32
L = 3

NTILES = 16
NP = 50176
RPT = NP // NTILES
ZCH = RPT // 8
EPT = 50176
EP = NTILES * EPT
EROWS = EP // 128
TROWS = EPT // 128
TR2 = TROWS // 2
NCH = 2
TRC = TROWS // NCH
EPC = EP // NCH
EPTC = EPT // NCH
TR2C = TR2 // NCH
TRC2 = TRC // 2
CROWS = EROWS // 2
CTROWS = CROWS // NTILES
RB = 1024
GRID = NP // RB

_SC_PARAMS = pltpu.CompilerParams(use_tc_tiling_on_sc=False)


def _edge_a_body(chunk, ei_hbm, g_hbm, za_hbm, zb_hbm,
                 idx_v, srcg_v, dstg_v, ra_v, rb_v, gsem, sasem, sbsem):
    c = lax.axis_index("c")
    s = lax.axis_index("s")

    offs = c * NP
    offd = 2 * NP + c * NP
    obase = c * EPC + s * EPTC

    def stage_fire(i):
        slot = i & 1
        pltpu.sync_copy(
            ei_hbm.at[pl.ds(s * TROWS + chunk * TRC + 2 * i, 2)],
            idx_v.at[slot])
        for p in range(2):
            for j in range(8):
                sj = pl.ds(j * 16, 16)
                srcg_v[slot, p, sj] = idx_v[slot, p, 0, sj] + offs
                dstg_v[slot, p, sj] = idx_v[slot, p, 1, sj] + offd
        sem = gsem.at[slot]
        for p in range(2):
            pltpu.async_copy(g_hbm.at[srcg_v.at[slot, p]],
                             ra_v.at[pl.ds(slot * 256 + p * 128, 128)], sem)
            pltpu.async_copy(g_hbm.at[dstg_v.at[slot, p]],
                             rb_v.at[pl.ds(slot * 256 + p * 128, 128)], sem)

    @pl.loop(0, TRC2 + 2)
    def _(i):
        @pl.when(i >= 2)
        def _():
            par = i & 1
            pltpu.make_async_copy(za_hbm.at[pl.ds(0, 256)],
                                  ra_v.at[pl.ds(0, 256)],
                                  sasem.at[par]).wait()
            pltpu.make_async_copy(zb_hbm.at[pl.ds(0, 256)],
                                  rb_v.at[pl.ds(0, 256)],
                                  sbsem.at[par]).wait()

        @pl.when(i < TRC2)
        def _():
            stage_fire(i)

        @pl.when(jnp.logical_and(i >= 1, i <= TRC2))
        def _():
            k = i - 1
            par = k & 1
            pltpu.make_async_copy(g_hbm.at[pl.ds(0, 512)],
                                  ra_v.at[pl.ds(0, 512)],
                                  gsem.at[par]).wait()
            orow = obase + k * 256
            pltpu.async_copy(ra_v.at[pl.ds(par * 256, 256)],
                             za_hbm.at[pl.ds(orow, 256)], sasem.at[par])
            pltpu.async_copy(rb_v.at[pl.ds(par * 256, 256)],
                             zb_hbm.at[pl.ds(orow, 256)], sbsem.at[par])


def _make_edge_a(chunk):
  return pl.kernel(
    functools.partial(_edge_a_body, chunk),
    out_type=[jax.ShapeDtypeStruct((2 * EPC, HH), jnp.float32),
              jax.ShapeDtypeStruct((2 * EPC, HH), jnp.float32)],
    mesh=plsc.VectorSubcoreMesh(core_axis_name="c", subcore_axis_name="s"),
    scratch_types=[
        pltpu.VMEM((2, 2, 2, 128), jnp.int32),
        pltpu.VMEM((2, 2, 128), jnp.int32),
        pltpu.VMEM((2, 2, 128), jnp.int32),
        pltpu.VMEM((512, HH), jnp.float32),
        pltpu.VMEM((512, HH), jnp.float32),
        pltpu.SemaphoreType.DMA((2,)),
        pltpu.SemaphoreType.DMA((2,)),
        pltpu.SemaphoreType.DMA((2,)),
    ],
    compiler_params=_SC_PARAMS,
  )


_edge_a1 = _make_edge_a(0)
_edge_a2 = _make_edge_a(1)


def _edge_b_body(ei_hbm, s1_hbm, s2_hbm, out_hbm,
                 idx_v, sv_v, s_sh, lsem, ssem):
    c = lax.axis_index("c")
    s = lax.axis_index("s")

    zero = jnp.zeros((1, 16), jnp.float32)

    @pl.loop(0, 256)
    def _(i):
        for j in range(HH // 16):
            sv_v.at[pl.ds(i, 1), pl.ds(j * 16, 16)][...] = zero
    for k in range(12):
        pltpu.sync_copy(sv_v.at[pl.ds(0, 256)],
                        s_sh.at[pl.ds(s * RPT + k * 256, 256)])
    pltpu.sync_copy(sv_v.at[pl.ds(0, RPT - 3072)],
                    s_sh.at[pl.ds(s * RPT + 3072, RPT - 3072)])
    plsc.subcore_barrier()

    ibase = c * EPC + s * EPTC

    @pl.loop(0, TR2 + 1)
    def _(i):
        @pl.when(i >= 2)
        def _():
            pltpu.make_async_copy(s1_hbm.at[pl.ds(0, 256)],
                                  sv_v.at[pl.ds(0, 256)],
                                  ssem.at[i & 1]).wait()

        @pl.when(i < TR2)
        def _():
            slot = i & 1
            pltpu.sync_copy(ei_hbm.at[s * TROWS + 2 * i, 1],
                            idx_v.at[slot, 0])
            pltpu.sync_copy(ei_hbm.at[s * TROWS + 2 * i + 1, 1],
                            idx_v.at[slot, 1])

            @pl.when(i < TR2C)
            def _():
                pltpu.async_copy(s1_hbm.at[pl.ds(ibase + i * 256, 256)],
                                 sv_v.at[pl.ds(slot * 256, 256)],
                                 lsem.at[slot])

            @pl.when(i >= TR2C)
            def _():
                pltpu.async_copy(
                    s2_hbm.at[pl.ds(ibase + (i - TR2C) * 256, 256)],
                    sv_v.at[pl.ds(slot * 256, 256)],
                    lsem.at[slot])

        @pl.when(jnp.logical_and(i >= 1, i <= TR2))
        def _():
            k = i - 1
            par = k & 1
            pltpu.make_async_copy(s1_hbm.at[pl.ds(0, 256)],
                                  sv_v.at[pl.ds(0, 256)],
                                  lsem.at[par]).wait()
            pltpu.async_copy(sv_v.at[pl.ds(par * 256, 128)],
                             s_sh.at[idx_v.at[par, 0]],
                             ssem.at[par], add=True)
            pltpu.async_copy(sv_v.at[pl.ds(par * 256 + 128, 128)],
                             s_sh.at[idx_v.at[par, 1]],
                             ssem.at[par], add=True)

    pltpu.make_async_copy(s1_hbm.at[pl.ds(0, 256)],
                          sv_v.at[pl.ds(0, 256)],
                          ssem.at[(TR2 - 1) & 1]).wait()

    plsc.subcore_barrier()
    pltpu.sync_copy(s_sh.at[pl.ds(s * RPT, RPT)],
                    out_hbm.at[pl.ds(c * NP + s * RPT, RPT)])


_edge_b = pl.kernel(
    _edge_b_body,
    out_type=jax.ShapeDtypeStruct((2 * NP, HH), jnp.float32),
    mesh=plsc.VectorSubcoreMesh(core_axis_name="c", subcore_axis_name="s"),
    scratch_types=[
        pltpu.VMEM((2, 2, 128), jnp.int32),
        pltpu.VMEM((512, HH), jnp.float32),
        pltpu.VMEM_SHARED((NP, HH), jnp.float32),
        pltpu.SemaphoreType.DMA((2,)),
        pltpu.SemaphoreType.DMA((2,)),
    ],
    compiler_params=_SC_PARAMS,
)


def _cnt_body(ei_hbm, out_hbm, idx_v, ones_v, z1_v, cnt_sh):
    c = lax.axis_index("c")
    s = lax.axis_index("s")
    one = jnp.ones((16,), jnp.float32)
    for j in range(8):
        ones_v[pl.ds(j * 16, 16)] = one
    zero = jnp.zeros((16,), jnp.float32)

    @pl.loop(0, RPT // 16)
    def _(i):
        z1_v[pl.ds(i * 16, 16)] = zero
    pltpu.sync_copy(z1_v, cnt_sh.at[pl.ds(s * RPT, RPT)])
    plsc.subcore_barrier()

    base_row = c * CROWS + s * CTROWS

    @pl.loop(0, CTROWS)
    def _(i):
        pltpu.sync_copy(ei_hbm.at[base_row + i, 1], idx_v.at[0])
        pltpu.sync_copy(ones_v, cnt_sh.at[idx_v.at[0]], add=True)

    plsc.subcore_barrier()
    pltpu.sync_copy(cnt_sh.at[pl.ds(s * RPT, RPT)],
                    out_hbm.at[pl.ds(c * NP + s * RPT, RPT)])


_cnt = pl.kernel(
    _cnt_body,
    out_type=jax.ShapeDtypeStruct((2 * NP,), jnp.float32),
    mesh=plsc.VectorSubcoreMesh(core_axis_name="c", subcore_axis_name="s"),
    scratch_types=[
        pltpu.VMEM((1, 128), jnp.int32),
        pltpu.VMEM((128,), jnp.float32),
        pltpu.VMEM((RPT,), jnp.float32),
        pltpu.VMEM_SHARED((NP,), jnp.float32),
    ],
    compiler_params=_SC_PARAMS,
)


def _prep_body(x_ref, wp_ref, bp_ref, wa_ref, wb_ref, h_ref, g_ref):
    x = x_ref[...]
    h = jnp.dot(x, wp_ref[...], preferred_element_type=jnp.float32) + bp_ref[...]
    h_ref[...] = h
    ga = jnp.dot(h, wa_ref[...], preferred_element_type=jnp.float32)
    gb = jnp.dot(h, wb_ref[...], preferred_element_type=jnp.float32)
    g_ref[0] = ga[:, :HH]
    g_ref[1] = ga[:, HH:]
    g_ref[2] = gb[:, :HH]
    g_ref[3] = gb[:, HH:]


def _update_body(last, *refs):
    if last:
        (h_ref, s_ref, cnt_ref, we2_ref, be2_ref, wn1a_ref, wn1b_ref,
         bn1_ref, wn2_ref, bn2_ref, h2_ref) = refs
    else:
        (h_ref, s_ref, cnt_ref, we2_ref, be2_ref, wn1a_ref, wn1b_ref,
         bn1_ref, wn2_ref, bn2_ref, wa_ref, wb_ref, h2_ref, g_ref) = refs
    h = h_ref[...]
    ssum = jnp.concatenate([s_ref[0], s_ref[1]], axis=1)
    cnt = cnt_ref[0] + cnt_ref[1]
    agg = (jnp.dot(ssum, we2_ref[...], preferred_element_type=jnp.float32)
           + cnt * be2_ref[...])
    pre = (jnp.dot(h, wn1a_ref[...], preferred_element_type=jnp.float32)
           + jnp.dot(agg, wn1b_ref[...], preferred_element_type=jnp.float32)
           + bn1_ref[...])
    t = pre * jax.nn.sigmoid(pre)
    h2 = h + jnp.dot(t, wn2_ref[...], preferred_element_type=jnp.float32) + bn2_ref[...]
    h2_ref[...] = h2
    if not last:
        ga = jnp.dot(h2, wa_ref[...], preferred_element_type=jnp.float32)
        gb = jnp.dot(h2, wb_ref[...], preferred_element_type=jnp.float32)
        g_ref[0] = ga[:, :HH]
        g_ref[1] = ga[:, HH:]
        g_ref[2] = gb[:, :HH]
        g_ref[3] = gb[:, HH:]


def _esilu_body(za_ref, zb_ref, bt_ref, s_ref):
    z = za_ref[...] + zb_ref[...] + bt_ref[...][:, None, :]
    s_ref[...] = z * jax.nn.sigmoid(z)


def _logits_body(h_ref, wr1_ref, br1_ref, wr2_ref, br2_ref, o_ref):
    h = h_ref[...]
    pre = jnp.dot(h, wr1_ref[...], preferred_element_type=jnp.float32) + br1_ref[...]
    t = pre * jax.nn.sigmoid(pre)
    o_ref[...] = jnp.sum(t * wr2_ref[...], axis=1, keepdims=True) + br2_ref[...]


def _softmax_body(l_ref, o_ref):
    r = lax.broadcasted_iota(jnp.int32, (NP // 128, 128), 0)
    cidx = lax.broadcasted_iota(jnp.int32, (NP // 128, 128), 1)
    flat = r * 128 + cidx
    logits = jnp.where(flat < N_NODES, l_ref[...], -jnp.inf)
    m = jnp.max(jnp.max(logits, axis=1, keepdims=True), axis=0, keepdims=True)
    e = jnp.exp(logits - m)
    ssum = jnp.sum(jnp.sum(e, axis=1, keepdims=True), axis=0, keepdims=True)
    o_ref[...] = e / ssum


_WSPEC = pl.BlockSpec((H, H), lambda i: (0, 0))
_BSPEC = pl.BlockSpec((1, H), lambda i: (0, 0))
_HSPEC = pl.BlockSpec((RB, H), lambda i: (i, 0))
_GSPEC = pl.BlockSpec((4, RB, HH), lambda i: (0, i, 0))

_prep = pl.pallas_call(
    _prep_body,
    grid=(GRID,),
    in_specs=[pl.BlockSpec((RB, 8), lambda i: (i, 0)),
              pl.BlockSpec((8, H), lambda i: (0, 0)),
              _BSPEC, _WSPEC, _WSPEC],
    out_specs=[_HSPEC, _GSPEC],
    out_shape=[jax.ShapeDtypeStruct((NP, H), jnp.float32),
               jax.ShapeDtypeStruct((4, NP, HH), jnp.float32)],
)

_update_specs = [
    _HSPEC,
    pl.BlockSpec((2, RB, HH), lambda i: (0, i, 0)),
    pl.BlockSpec((2, RB, 1), lambda i: (0, i, 0)),
    _WSPEC, _BSPEC, _WSPEC, _WSPEC, _BSPEC, _WSPEC, _BSPEC,
]

_update_mid = pl.pallas_call(
    functools.partial(_update_body, False),
    grid=(GRID,),
    in_specs=_update_specs + [_WSPEC, _WSPEC],
    out_specs=[_HSPEC, _GSPEC],
    out_shape=[jax.ShapeDtypeStruct((NP, H), jnp.float32),
               jax.ShapeDtypeStruct((4, NP, HH), jnp.float32)],
)

_update_last = pl.pallas_call(
    functools.partial(_update_body, True),
    grid=(GRID,),
    in_specs=_update_specs,
    out_specs=[_HSPEC],
    out_shape=[jax.ShapeDtypeStruct((NP, H), jnp.float32)],
)

ERB = 2048
EGRID = EPC // 4 // ERB

_ESPEC = pl.BlockSpec((2, ERB, 128), lambda i: (0, i, 0))

_esilu = pl.pallas_call(
    _esilu_body,
    grid=(EGRID,),
    in_specs=[_ESPEC, _ESPEC, pl.BlockSpec((2, 128), lambda i: (0, 0))],
    out_specs=[_ESPEC],
    out_shape=[jax.ShapeDtypeStruct((2, EPC // 4, 128), jnp.float32)],
)

_logits = pl.pallas_call(
    _logits_body,
    grid=(GRID,),
    in_specs=[_HSPEC, _WSPEC, _BSPEC, _BSPEC,
              pl.BlockSpec((1, 1), lambda i: (0, 0))],
    out_specs=[pl.BlockSpec((RB, 1), lambda i: (i, 0))],
    out_shape=[jax.ShapeDtypeStruct((NP, 1), jnp.float32)],
)

_softmax = pl.pallas_call(
    _softmax_body,
    out_shape=jax.ShapeDtypeStruct((NP // 128, 128), jnp.float32),
)


def kernel(node_feats, edge_index, Wp, bp, We1, be1, We2, be2,
           Wn1, bn1, Wn2, bn2, Wr1, br1, Wr2, br2):
    f32 = jnp.float32
    xp = jnp.zeros((NP, 8), f32).at[:N_NODES, :5].set(node_feats)
    wp8 = jnp.zeros((8, H), f32).at[:5].set(Wp)
    src = edge_index[0]
    dst = edge_index[1]
    pad = (N_NODES
           + (jnp.arange(EP - N_EDGES, dtype=jnp.int32) % (NP - N_NODES)))
    srcp = jnp.concatenate([src, pad]).reshape(EROWS, 128)
    dstp = jnp.concatenate([dst, pad]).reshape(EROWS, 128)
    ei = jnp.stack([srcp, dstp], axis=1)

    cnt = _cnt(ei).reshape(2, NP, 1)

    h, g = _prep(xp, wp8, bp.reshape(1, H), We1[0][:H], We1[0][H:])

    for l in range(L):
        gflat = g.reshape(4 * NP, HH)
        bt = jnp.tile(be1[l].reshape(2, HH), (1, 4))
        za1, zb1 = _edge_a1(ei, gflat)
        za2, zb2 = _edge_a2(ei, gflat)
        (s1,) = _esilu(za1.reshape(2, EPC // 4, 128),
                       zb1.reshape(2, EPC // 4, 128), bt)
        (s2,) = _esilu(za2.reshape(2, EPC // 4, 128),
                       zb2.reshape(2, EPC // 4, 128), bt)
        s_acc = _edge_b(ei, s1.reshape(2 * EPC, HH), s2.reshape(2 * EPC, HH))
        s3 = s_acc.reshape(2, NP, HH)
        args = (h, s3, cnt, We2[l], be2[l].reshape(1, H),
                Wn1[l][:H], Wn1[l][H:], bn1[l].reshape(1, H),
                Wn2[l], bn2[l].reshape(1, H))
        if l < L - 1:
            h, g = _update_mid(*args, We1[l + 1][:H], We1[l + 1][H:])
        else:
            (h,) = _update_last(*args)

    (logits,) = _logits(h, Wr1, br1.reshape(1, H), Wr2.reshape(1, H),
                        br2.reshape(1, 1))
    probs = _softmax(logits.reshape(NP // 128, 128))
    return probs.reshape(NP)[:N_NODES]
